# Initial kernel scaffold; baseline (speedup 1.0000x reference)
import jax
import jax.numpy as jnp
from jax.experimental import pallas as pl


def kernel(x, edge_index, batch, W_in, b_in, W_c0, b_c0, g0, be0, W_c1, b_c1, g1, be1, W_c2, b_c2, g2, be2, fc_W1, fc_b1, fc_W2, fc_b2, fc_W3, fc_b3):
    raise NotImplementedError("write your pallas kernel here")


# speedup vs baseline: 1.3436x; 1.3436x over previous
import functools

import jax
import jax.numpy as jnp
from jax.experimental import pallas as pl
from jax.experimental.pallas import tpu as pltpu

N = 100000
E = 400000
HID = 64
NPG = 5
ROWS_BLK = 2000
HEAD_BLK = 2000


def _gelu(t):
    return 0.5 * t * (1.0 + jax.lax.erf(t * 0.7071067811865475))


def _in_proj_body(x_ref, w_ref, b_ref, o_ref):
    o_ref[...] = (
        jnp.dot(x_ref[...], w_ref[...], preferred_element_type=jnp.float32)
        + b_ref[...]
    )


def _in_proj(x_p, w_p, b):
    grid = (N // ROWS_BLK,)
    return pl.pallas_call(
        _in_proj_body,
        grid=grid,
        in_specs=[
            pl.BlockSpec((ROWS_BLK, 8), lambda i: (i, 0)),
            pl.BlockSpec((8, HID), lambda i: (0, 0)),
            pl.BlockSpec((1, HID), lambda i: (0, 0)),
        ],
        out_specs=pl.BlockSpec((ROWS_BLK, HID), lambda i: (i, 0)),
        out_shape=jax.ShapeDtypeStruct((N, HID), jnp.float32),
    )(x_p, w_p, b)


def _matmul_body(h_ref, w_ref, o_ref):
    o_ref[...] = jnp.dot(h_ref[...], w_ref[...], preferred_element_type=jnp.float32)


def _matmul(h, w):
    grid = (N // ROWS_BLK,)
    return pl.pallas_call(
        _matmul_body,
        grid=grid,
        in_specs=[
            pl.BlockSpec((ROWS_BLK, HID), lambda i: (i, 0)),
            pl.BlockSpec((HID, HID), lambda i: (0, 0)),
        ],
        out_specs=pl.BlockSpec((ROWS_BLK, HID), lambda i: (i, 0)),
        out_shape=jax.ShapeDtypeStruct((N, HID), jnp.float32),
    )(h, w)


def _post_body(h_ref, agg_ref, m_ref, d2_ref, bc_ref, g_ref, be_ref, o_ref):
    agg = agg_ref[...] + d2_ref[...] * m_ref[...] + bc_ref[...]
    mu = jnp.mean(agg, axis=-1, keepdims=True)
    var = jnp.mean((agg - mu) * (agg - mu), axis=-1, keepdims=True)
    ln = (agg - mu) * jax.lax.rsqrt(var + 1e-5) * g_ref[...] + be_ref[...]
    o_ref[...] = h_ref[...] + _gelu(ln)


def _post(h, agg_e, m, d2, bc, g, be):
    grid = (N // ROWS_BLK,)
    return pl.pallas_call(
        _post_body,
        grid=grid,
        in_specs=[
            pl.BlockSpec((ROWS_BLK, HID), lambda i: (i, 0)),
            pl.BlockSpec((ROWS_BLK, HID), lambda i: (i, 0)),
            pl.BlockSpec((ROWS_BLK, HID), lambda i: (i, 0)),
            pl.BlockSpec((ROWS_BLK, 1), lambda i: (i, 0)),
            pl.BlockSpec((1, HID), lambda i: (0, 0)),
            pl.BlockSpec((1, HID), lambda i: (0, 0)),
            pl.BlockSpec((1, HID), lambda i: (0, 0)),
        ],
        out_specs=pl.BlockSpec((ROWS_BLK, HID), lambda i: (i, 0)),
        out_shape=jax.ShapeDtypeStruct((N, HID), jnp.float32),
    )(h, agg_e, m, d2, bc, g, be)


def _head_body(z_ref, w1_ref, b1_ref, w2_ref, b2_ref, w3_ref, b3_ref, o_ref):
    z = _gelu(
        jnp.dot(z_ref[...], w1_ref[...], preferred_element_type=jnp.float32)
        + b1_ref[...]
    )
    z = _gelu(
        jnp.dot(z, w2_ref[...], preferred_element_type=jnp.float32) + b2_ref[...]
    )
    o_ref[...] = (
        jnp.dot(z, w3_ref[...], preferred_element_type=jnp.float32) + b3_ref[...]
    )


def _head(z, w1, b1, w2, b2, w3_p, b3_p):
    bs = N // NPG
    grid = (bs // HEAD_BLK,)
    return pl.pallas_call(
        _head_body,
        grid=grid,
        in_specs=[
            pl.BlockSpec((HEAD_BLK, HID * NPG), lambda i: (i, 0)),
            pl.BlockSpec((HID * NPG, HID * 2), lambda i: (0, 0)),
            pl.BlockSpec((1, HID * 2), lambda i: (0, 0)),
            pl.BlockSpec((HID * 2, HID), lambda i: (0, 0)),
            pl.BlockSpec((1, HID), lambda i: (0, 0)),
            pl.BlockSpec((HID, 128), lambda i: (0, 0)),
            pl.BlockSpec((1, 128), lambda i: (0, 0)),
        ],
        out_specs=pl.BlockSpec((HEAD_BLK, 128), lambda i: (i, 0)),
        out_shape=jax.ShapeDtypeStruct((bs, 128), jnp.float32),
    )(z, w1, b1, w2, b2, w3_p, b3_p)


def kernel(x, edge_index, batch, W_in, b_in, W_c0, b_c0, g0, be0, W_c1, b_c1, g1,
           be1, W_c2, b_c2, g2, be2, fc_W1, fc_b1, fc_W2, fc_b2, fc_W3, fc_b3):
    src = edge_index[0]
    dst = edge_index[1]

    deg = jnp.ones((N,), jnp.float32).at[dst].add(1.0)
    dis = jax.lax.rsqrt(deg)
    d2 = (dis * dis).reshape(N, 1)
    norm = dis[src] * dis[dst]

    x_p = jnp.pad(x, ((0, 0), (0, 5)))
    w_in_p = jnp.pad(W_in, ((0, 5), (0, 0)))
    h = _in_proj(x_p, w_in_p, b_in.reshape(1, HID))

    for Wc, bc, g, be in (
        (W_c0, b_c0, g0, be0),
        (W_c1, b_c1, g1, be1),
        (W_c2, b_c2, g2, be2),
    ):
        m = _matmul(h, Wc)
        msg = norm[:, None] * m[src]
        agg_e = jnp.zeros((N, HID), jnp.float32).at[dst].add(msg)
        h = _post(h, agg_e, m, d2, bc.reshape(1, HID), g.reshape(1, HID),
                  be.reshape(1, HID))

    z = h.reshape(N // NPG, HID * NPG)
    w3_p = jnp.pad(fc_W3, ((0, 0), (0, 123)))
    b3_p = jnp.pad(fc_b3, ((0, 123))).reshape(1, 128)
    out = _head(z, fc_W1, fc_b1.reshape(1, HID * 2), fc_W2,
                fc_b2.reshape(1, HID), w3_p, b3_p)
    return out[:, :5]
